# Initial kernel scaffold; baseline (speedup 1.0000x reference)
import jax
import jax.numpy as jnp
from jax.experimental import pallas as pl


def kernel(inputs):
    raise NotImplementedError("write your pallas kernel here")


# speedup vs baseline: 3.3178x; 3.3178x over previous
import numpy as np

import jax
import jax.numpy as jnp
from jax import lax
from jax.experimental import pallas as pl
from jax.experimental.pallas import tpu as pltpu
from jax.experimental.pallas import tpu_sc as plsc

_B, _D = 64, 8192
_K = 1639
_L = 16
_NV = _D // _L
_NC, _NS = 2, 16
_NW = _NC * _NS
_RPW = _B // _NW
_IMIN = np.int32(-2147483648)


def _kwta_body(x_hbm, o_hbm, x_v, ku_v, hist_v):
    wid = lax.axis_index("s") * _NC + lax.axis_index("c")
    iota = lax.iota(jnp.int32, _L)
    ones = jnp.ones((_L,), jnp.int32)
    zeros = jnp.zeros((_L,), jnp.int32)
    for r in range(_RPW):
        row = wid * _RPW + r
        pltpu.sync_copy(x_hbm.at[row], x_v)

        def key_body(i, carry):
            x = x_v[pl.ds(i * _L, _L)]
            u = lax.bitcast_convert_type(x, jnp.int32)
            ku_v[pl.ds(i * _L, _L)] = jnp.where(u >= 0, u ^ _IMIN, ~u)
            return carry
        lax.fori_loop(0, _NV, key_body, jnp.int32(0))

        kprime = jnp.int32(_K)
        prefix = jnp.int32(0)
        pmask = jnp.int32(0)
        for p in range(4):
            shift = 24 - 8 * p
            for j in range(16):
                hist_v[pl.ds(j * _L, _L)] = zeros

            def hist_body(i, carry, shift=shift, prefix=prefix, pmask=pmask):
                ku = ku_v[pl.ds(i * _L, _L)]
                match = (ku & pmask) == prefix
                digit = lax.shift_right_logical(ku, shift) & 0xFF
                plsc.addupdate_scatter(hist_v, [digit], ones, mask=match)
                return carry
            lax.fori_loop(0, _NV, hist_body, jnp.int32(0))

            higher = jnp.int32(0)
            found = jnp.int32(0)
            dstar = jnp.int32(0)
            gt = jnp.int32(0)
            for j in range(15, -1, -1):
                v = hist_v[pl.ds(j * _L, _L)]
                sv = jnp.flip(jnp.cumsum(jnp.flip(v))) + higher
                m = sv >= kprime
                cnt = jnp.sum(m.astype(jnp.int32))
                sel = iota == (cnt - 1)
                suffix_at = jnp.sum(jnp.where(sel, sv, 0))
                bin_at = jnp.sum(jnp.where(sel, v, 0))
                use = (found == 0) & (cnt > 0)
                dstar = jnp.where(use, j * _L + cnt - 1, dstar)
                gt = jnp.where(use, suffix_at - bin_at, gt)
                found = jnp.where(cnt > 0, jnp.int32(1), found)
                higher = higher + jnp.sum(v)
            kprime = kprime - gt
            prefix = prefix | lax.shift_left(dstar, jnp.int32(shift))
            pmask = pmask | lax.shift_left(jnp.int32(0xFF), jnp.int32(shift))

        thresh = prefix ^ _IMIN

        def mask_body(i, carry, thresh=thresh):
            ks = ku_v[pl.ds(i * _L, _L)] ^ _IMIN
            x = x_v[pl.ds(i * _L, _L)]
            x_v[pl.ds(i * _L, _L)] = jnp.where(ks >= thresh, x, 0.0)
            return carry
        lax.fori_loop(0, _NV, mask_body, jnp.int32(0))

        pltpu.sync_copy(x_v, o_hbm.at[row])


def kernel(inputs):
    mesh = plsc.VectorSubcoreMesh(
        core_axis_name="c", subcore_axis_name="s", num_cores=_NC,
        num_subcores=_NS)
    f = pl.kernel(
        _kwta_body,
        out_type=jax.ShapeDtypeStruct((_B, _D), jnp.float32),
        mesh=mesh,
        scratch_types=[
            pltpu.VMEM((_D,), jnp.float32),
            pltpu.VMEM((_D,), jnp.int32),
            pltpu.VMEM((256,), jnp.int32),
        ],
        compiler_params=pltpu.CompilerParams(needs_layout_passes=False),
    )
    return f(inputs)


# speedup vs baseline: 4.0185x; 1.2112x over previous
import numpy as np

import jax
import jax.numpy as jnp
from jax import lax
from jax.experimental import pallas as pl
from jax.experimental.pallas import tpu as pltpu
from jax.experimental.pallas import tpu_sc as plsc

_B, _D = 64, 8192
_K = 1639
_L = 16
_NV = _D // _L
_U = 8
_NC, _NS = 2, 16
_NW = _NC * _NS
_RPW = _B // _NW
_IMIN = np.int32(-2147483648)


def _kwta_body(x_hbm, o_hbm, x_v, ku_v, hist_v):
    wid = lax.axis_index("s") * _NC + lax.axis_index("c")
    iota = lax.iota(jnp.int32, _L)
    ones = jnp.ones((_L,), jnp.int32)
    zeros = jnp.zeros((_L,), jnp.int32)
    for r in range(_RPW):
        row = wid * _RPW + r
        pltpu.sync_copy(x_hbm.at[row], x_v)

        for j in range(16):
            hist_v[pl.ds(j * _L, _L)] = zeros

        def p1_body(i, carry):
            for u in range(_U):
                s = i * (_L * _U) + u * _L
                x = x_v[pl.ds(s, _L)]
                b = lax.bitcast_convert_type(x, jnp.int32)
                ku = jnp.where(b >= 0, b ^ _IMIN, ~b)
                ku_v[pl.ds(s, _L)] = ku
                digit = lax.shift_right_logical(ku, 24)
                plsc.addupdate_scatter(hist_v, [digit], ones)
            return carry
        lax.fori_loop(0, _NV // _U, p1_body, jnp.int32(0))

        kprime = jnp.int32(_K)
        prefix = jnp.int32(0)
        pmask = jnp.int32(0)
        for p in range(4):
            shift = 24 - 8 * p
            if p > 0:
                def hp_body(i, carry, shift=shift, prefix=prefix,
                            pmask=pmask):
                    for u in range(_U):
                        s = i * (_L * _U) + u * _L
                        ku = ku_v[pl.ds(s, _L)]
                        match = (ku & pmask) == prefix
                        if shift:
                            digit = lax.shift_right_logical(ku, shift) & 0xFF
                        else:
                            digit = ku & 0xFF
                        plsc.addupdate_scatter(hist_v, [digit], ones,
                                               mask=match)
                    return carry
                lax.fori_loop(0, _NV // _U, hp_body, jnp.int32(0))

            gs = [jnp.sum(hist_v[pl.ds(j * _L, _L)]) for j in range(16)]
            higher = jnp.int32(0)
            found = jnp.int32(0)
            jstar = jnp.int32(0)
            hab = jnp.int32(0)
            for j in range(15, -1, -1):
                tot = higher + gs[j]
                hit = tot >= kprime
                use = (found == 0) & hit
                jstar = jnp.where(use, jnp.int32(j), jstar)
                hab = jnp.where(use, higher, hab)
                found = jnp.where(hit, jnp.int32(1), found)
                higher = tot
            v = hist_v[pl.ds(jstar * _L, _L)]
            sv = jnp.flip(jnp.cumsum(jnp.flip(v))) + hab
            m = sv >= kprime
            cnt = jnp.sum(m.astype(jnp.int32))
            sel = iota == (cnt - 1)
            suffix_at = jnp.sum(jnp.where(sel, sv, 0))
            bin_at = jnp.sum(jnp.where(sel, v, 0))
            dstar = jstar * _L + cnt - 1
            kprime = kprime - (suffix_at - bin_at)
            prefix = prefix | lax.shift_left(dstar, jnp.int32(shift))
            pmask = pmask | lax.shift_left(jnp.int32(0xFF), jnp.int32(shift))
            if p < 3:
                for j in range(16):
                    hist_v[pl.ds(j * _L, _L)] = zeros

        thresh = prefix ^ _IMIN

        def mask_body(i, carry, thresh=thresh):
            for u in range(_U):
                s = i * (_L * _U) + u * _L
                ks = ku_v[pl.ds(s, _L)] ^ _IMIN
                x = x_v[pl.ds(s, _L)]
                x_v[pl.ds(s, _L)] = jnp.where(ks >= thresh, x, 0.0)
            return carry
        lax.fori_loop(0, _NV // _U, mask_body, jnp.int32(0))

        pltpu.sync_copy(x_v, o_hbm.at[row])


def kernel(inputs):
    mesh = plsc.VectorSubcoreMesh(
        core_axis_name="c", subcore_axis_name="s", num_cores=_NC,
        num_subcores=_NS)
    f = pl.kernel(
        _kwta_body,
        out_type=jax.ShapeDtypeStruct((_B, _D), jnp.float32),
        mesh=mesh,
        scratch_types=[
            pltpu.VMEM((_D,), jnp.float32),
            pltpu.VMEM((_D,), jnp.int32),
            pltpu.VMEM((256,), jnp.int32),
        ],
        compiler_params=pltpu.CompilerParams(needs_layout_passes=False),
    )
    return f(inputs)


# speedup vs baseline: 7.1443x; 1.7779x over previous
import numpy as np

import jax
import jax.numpy as jnp
from jax import lax
from jax.experimental import pallas as pl
from jax.experimental.pallas import tpu as pltpu
from jax.experimental.pallas import tpu_sc as plsc

_B, _D = 64, 8192
_K = 1639
_L = 16
_NV = _D // _L
_U = 8
_NC, _NS = 2, 16
_NW = _NC * _NS
_RPW = _B // _NW
_IMIN = np.int32(-2147483648)


def _kwta_body(x_hbm, o_hbm, x_v, ku_v, hist_v):
    wid = lax.axis_index("s") * _NC + lax.axis_index("c")
    iota = lax.iota(jnp.int32, _L)
    ones = jnp.ones((_L,), jnp.int32)
    zeros = jnp.zeros((_L,), jnp.int32)
    for r in range(_RPW):
        row = wid * _RPW + r
        pltpu.sync_copy(x_hbm.at[row], x_v)

        for j in range(16):
            hist_v[pl.ds(j * _L, _L)] = zeros

        @plsc.parallel_loop(0, _NV, 1, unroll=_U)
        def _(i):
            s = i * _L
            x = x_v[pl.ds(s, _L)]
            b = lax.bitcast_convert_type(x, jnp.int32)
            ku = jnp.where(b >= 0, b ^ _IMIN, ~b)
            ku_v[pl.ds(s, _L)] = ku
            digit = lax.shift_right_logical(ku, 24)
            plsc.addupdate_scatter(hist_v, [digit], ones)

        kprime = jnp.int32(_K)
        prefix = jnp.int32(0)
        pmask = jnp.int32(0)
        for p in range(4):
            shift = 24 - 8 * p
            if p > 0:
                @plsc.parallel_loop(0, _NV, 1, unroll=_U)
                def _(i, shift=shift, prefix=prefix, pmask=pmask):
                    s = i * _L
                    ku = ku_v[pl.ds(s, _L)]
                    match = (ku & pmask) == prefix
                    if shift:
                        digit = lax.shift_right_logical(ku, shift) & 0xFF
                    else:
                        digit = ku & 0xFF
                    plsc.addupdate_scatter(hist_v, [digit], ones,
                                           mask=match)

            gs = [jnp.sum(hist_v[pl.ds(j * _L, _L)]) for j in range(16)]
            higher = jnp.int32(0)
            found = jnp.int32(0)
            jstar = jnp.int32(0)
            hab = jnp.int32(0)
            for j in range(15, -1, -1):
                tot = higher + gs[j]
                hit = tot >= kprime
                use = (found == 0) & hit
                jstar = jnp.where(use, jnp.int32(j), jstar)
                hab = jnp.where(use, higher, hab)
                found = jnp.where(hit, jnp.int32(1), found)
                higher = tot
            v = hist_v[pl.ds(jstar * _L, _L)]
            sv = jnp.flip(jnp.cumsum(jnp.flip(v))) + hab
            m = sv >= kprime
            cnt = jnp.sum(m.astype(jnp.int32))
            sel = iota == (cnt - 1)
            suffix_at = jnp.sum(jnp.where(sel, sv, 0))
            bin_at = jnp.sum(jnp.where(sel, v, 0))
            dstar = jstar * _L + cnt - 1
            kprime = kprime - (suffix_at - bin_at)
            prefix = prefix | lax.shift_left(dstar, jnp.int32(shift))
            pmask = pmask | lax.shift_left(jnp.int32(0xFF), jnp.int32(shift))
            if p < 3:
                for j in range(16):
                    hist_v[pl.ds(j * _L, _L)] = zeros

        thresh = prefix ^ _IMIN

        @plsc.parallel_loop(0, _NV, 1, unroll=_U)
        def _(i, thresh=thresh):
            s = i * _L
            ks = ku_v[pl.ds(s, _L)] ^ _IMIN
            x = x_v[pl.ds(s, _L)]
            x_v[pl.ds(s, _L)] = jnp.where(ks >= thresh, x, 0.0)

        pltpu.sync_copy(x_v, o_hbm.at[row])


def kernel(inputs):
    mesh = plsc.VectorSubcoreMesh(
        core_axis_name="c", subcore_axis_name="s", num_cores=_NC,
        num_subcores=_NS)
    f = pl.kernel(
        _kwta_body,
        out_type=jax.ShapeDtypeStruct((_B, _D), jnp.float32),
        mesh=mesh,
        scratch_types=[
            pltpu.VMEM((_D,), jnp.float32),
            pltpu.VMEM((_D,), jnp.int32),
            pltpu.VMEM((256,), jnp.int32),
        ],
        compiler_params=pltpu.CompilerParams(needs_layout_passes=False),
    )
    return f(inputs)
